# baseline (speedup 1.0000x reference)
import functools

import jax
import jax.numpy as jnp
from jax import lax
from jax.experimental import pallas as pl
from jax.experimental.pallas import tpu as pltpu
from jax.experimental.pallas import tpu_sc as plsc

_B = 16384
_NC = 1
_NS = 16
_NW = _NC * _NS
_PER_W = _B // _NW
_L = 16
_NVEC = _PER_W // _L


def _sc_body(idx_hbm, emb_hbm, scale_hbm, bias_hbm, out_hbm,
             idx_v, tab_v, out_v, sem_idx, sem_tab):
    wid = lax.axis_index("s") * _NC + lax.axis_index("c")
    base = wid * _PER_W

    idx_cp = pltpu.async_copy(idx_hbm.at[pl.ds(base, _PER_W)], idx_v, sem_idx)
    emb_cp = pltpu.async_copy(emb_hbm, tab_v.at[pl.ds(0, 10)], sem_tab)
    scl_cp = pltpu.async_copy(scale_hbm, tab_v.at[pl.ds(16, 1)], sem_tab)
    bia_cp = pltpu.async_copy(bias_hbm, tab_v.at[pl.ds(24, 1)], sem_tab)
    emb_cp.wait()
    scl_cp.wait()
    bia_cp.wait()

    zeros = jnp.zeros((_L,), jnp.int32)
    scale = jnp.take_along_axis(tab_v[pl.ds(16, _L)], zeros, axis=0)
    bias = jnp.take_along_axis(tab_v[pl.ds(24, _L)], zeros, axis=0)
    lut = tab_v[pl.ds(0, _L)] * scale + bias

    idx_cp.wait()

    def _step(i, carry):
        off = i * _L
        iv = idx_v[pl.ds(off, _L)]
        out_v[pl.ds(off, _L)] = jnp.take_along_axis(lut, iv, axis=0)
        return carry

    lax.fori_loop(0, _NVEC, _step, 0, unroll=4)

    pltpu.sync_copy(out_v, out_hbm.at[pl.ds(base, _PER_W)])


@jax.jit
def _run(idx, emb, scale, bias):
    mesh = plsc.VectorSubcoreMesh(
        core_axis_name="c", subcore_axis_name="s", num_cores=_NC)
    k = functools.partial(
        pl.kernel,
        out_type=jax.ShapeDtypeStruct((_B,), jnp.float32),
        mesh=mesh,
        scratch_types=[
            pltpu.VMEM((_PER_W,), jnp.int32),
            pltpu.VMEM((40,), jnp.float32),
            pltpu.VMEM((_PER_W,), jnp.float32),
            pltpu.SemaphoreType.DMA,
            pltpu.SemaphoreType.DMA,
        ],
    )(_sc_body)
    return k(idx, emb, scale, bias)


def kernel(inputs, embeddings, dense_kernel, dense_bias):
    idx = inputs.reshape(_B).astype(jnp.int32)
    out = _run(idx, embeddings.reshape(10), dense_kernel.reshape(1),
               dense_bias.reshape(1))
    return out.reshape(_B, 1, 1)


# speedup vs baseline: 1.0088x; 1.0088x over previous
import functools

import jax
import jax.numpy as jnp
from jax import lax
from jax.experimental import pallas as pl
from jax.experimental.pallas import tpu as pltpu
from jax.experimental.pallas import tpu_sc as plsc

_B = 16384
_NC = 1
_NS = 16
_NW = _NC * _NS
_PER_W = _B // _NW
_L = 16
_NVEC = _PER_W // _L


def _sc_body(idx_hbm, emb_hbm, scale_hbm, bias_hbm, out_hbm,
             idx_v, tab_v, out_v, sem_idx, sem_tab):
    wid = lax.axis_index("s") * _NC + lax.axis_index("c")
    base = wid * _PER_W

    idx_cp = pltpu.async_copy(idx_hbm.at[pl.ds(base, _PER_W)], idx_v, sem_idx)
    emb_cp = pltpu.async_copy(emb_hbm, tab_v.at[pl.ds(0, 10)], sem_tab)
    scl_cp = pltpu.async_copy(scale_hbm, tab_v.at[pl.ds(16, 1)], sem_tab)
    bia_cp = pltpu.async_copy(bias_hbm, tab_v.at[pl.ds(24, 1)], sem_tab)
    emb_cp.wait()
    scl_cp.wait()
    bia_cp.wait()

    zeros = jnp.zeros((_L,), jnp.int32)
    scale = jnp.take_along_axis(tab_v[pl.ds(16, _L)], zeros, axis=0)
    bias = jnp.take_along_axis(tab_v[pl.ds(24, _L)], zeros, axis=0)
    lut = tab_v[pl.ds(0, _L)] * scale + bias

    idx_cp.wait()

    half = _PER_W // 2
    for i in range(_NVEC // 2):
        iv = idx_v[pl.ds(i * _L, _L)]
        out_v[pl.ds(i * _L, _L)] = jnp.take_along_axis(lut, iv, axis=0)
    out_cp0 = pltpu.async_copy(
        out_v.at[pl.ds(0, half)], out_hbm.at[pl.ds(base, half)], sem_idx)
    for i in range(_NVEC // 2, _NVEC):
        iv = idx_v[pl.ds(i * _L, _L)]
        out_v[pl.ds(i * _L, _L)] = jnp.take_along_axis(lut, iv, axis=0)
    out_cp1 = pltpu.async_copy(
        out_v.at[pl.ds(half, half)], out_hbm.at[pl.ds(base + half, half)],
        sem_tab)
    out_cp0.wait()
    out_cp1.wait()


@jax.jit
def _run(idx, emb, scale, bias):
    mesh = plsc.VectorSubcoreMesh(
        core_axis_name="c", subcore_axis_name="s", num_cores=_NC)
    k = functools.partial(
        pl.kernel,
        out_type=jax.ShapeDtypeStruct((_B,), jnp.float32),
        mesh=mesh,
        scratch_types=[
            pltpu.VMEM((_PER_W,), jnp.int32),
            pltpu.VMEM((40,), jnp.float32),
            pltpu.VMEM((_PER_W,), jnp.float32),
            pltpu.SemaphoreType.DMA,
            pltpu.SemaphoreType.DMA,
        ],
    )(_sc_body)
    return k(idx, emb, scale, bias)


def kernel(inputs, embeddings, dense_kernel, dense_bias):
    idx = inputs.reshape(_B).astype(jnp.int32)
    out = _run(idx, embeddings.reshape(10), dense_kernel.reshape(1),
               dense_bias.reshape(1))
    return out.reshape(_B, 1, 1)


# speedup vs baseline: 1.0213x; 1.0124x over previous
import functools

import jax
import jax.numpy as jnp
from jax import lax
from jax.experimental import pallas as pl
from jax.experimental.pallas import tpu as pltpu
from jax.experimental.pallas import tpu_sc as plsc

_B = 16384
_NC = 1
_NS = 16
_NW = _NC * _NS
_PER_W = _B // _NW
_L = 16
_NVEC = _PER_W // _L


def _sc_body(idx_hbm, emb_hbm, scale_hbm, bias_hbm, out_hbm,
             idx_v, tab_v, out_v, sem_idx, sem_tab):
    wid = lax.axis_index("s") * _NC + lax.axis_index("c")
    base = wid * _PER_W

    idx_cp = pltpu.async_copy(idx_hbm.at[pl.ds(base, _PER_W)], idx_v, sem_idx)
    emb_cp = pltpu.async_copy(emb_hbm, tab_v.at[pl.ds(0, 10)], sem_tab)
    scl_cp = pltpu.async_copy(scale_hbm, tab_v.at[pl.ds(16, 1)], sem_tab)
    bia_cp = pltpu.async_copy(bias_hbm, tab_v.at[pl.ds(24, 1)], sem_tab)
    emb_cp.wait()
    scl_cp.wait()
    bia_cp.wait()

    zeros = jnp.zeros((_L,), jnp.int32)
    scale = jnp.take_along_axis(tab_v[pl.ds(16, _L)], zeros, axis=0)
    bias = jnp.take_along_axis(tab_v[pl.ds(24, _L)], zeros, axis=0)
    lut = tab_v[pl.ds(0, _L)] * scale + bias

    idx_cp.wait()

    half = _PER_W // 2

    @plsc.parallel_loop(0, half, _L, unroll=8)
    def _first(off):
        iv = idx_v[pl.ds(off, _L)]
        out_v[pl.ds(off, _L)] = jnp.take_along_axis(lut, iv, axis=0)

    out_cp0 = pltpu.async_copy(
        out_v.at[pl.ds(0, half)], out_hbm.at[pl.ds(base, half)], sem_idx)

    @plsc.parallel_loop(half, _PER_W, _L, unroll=8)
    def _second(off):
        iv = idx_v[pl.ds(off, _L)]
        out_v[pl.ds(off, _L)] = jnp.take_along_axis(lut, iv, axis=0)

    out_cp1 = pltpu.async_copy(
        out_v.at[pl.ds(half, half)], out_hbm.at[pl.ds(base + half, half)],
        sem_tab)
    out_cp0.wait()
    out_cp1.wait()


@jax.jit
def _run(idx, emb, scale, bias):
    mesh = plsc.VectorSubcoreMesh(
        core_axis_name="c", subcore_axis_name="s", num_cores=_NC)
    k = functools.partial(
        pl.kernel,
        out_type=jax.ShapeDtypeStruct((_B,), jnp.float32),
        mesh=mesh,
        scratch_types=[
            pltpu.VMEM((_PER_W,), jnp.int32),
            pltpu.VMEM((40,), jnp.float32),
            pltpu.VMEM((_PER_W,), jnp.float32),
            pltpu.SemaphoreType.DMA,
            pltpu.SemaphoreType.DMA,
        ],
    )(_sc_body)
    return k(idx, emb, scale, bias)


def kernel(inputs, embeddings, dense_kernel, dense_bias):
    idx = inputs.reshape(_B).astype(jnp.int32)
    out = _run(idx, embeddings.reshape(10), dense_kernel.reshape(1),
               dense_bias.reshape(1))
    return out.reshape(_B, 1, 1)
